# baseline (speedup 1.0000x reference)
import functools

import jax
import jax.numpy as jnp
from jax import lax
from jax.experimental import pallas as pl
from jax.experimental.pallas import tpu as pltpu
from jax.experimental.pallas import tpu_sc as plsc

_NUM_CLASSES = 100000
_D = 128
_B = 1024
_W = 20
_S = 16
_N = _B * _W
_NC = 2
_NSC = 16
_NW = _NC * _NSC
_RPT = _N // _NW
_RB = 16
_NBLK = _RPT // _RB
_L = 16
_BLK_W = (_S + 1) * _RB * _L
_NBLK_G = _N // _RB

_TC_STEPS = 20
_TC_ROWS = _NBLK_G * _BLK_W // 128 // _TC_STEPS


def _sc_scores(in_tab, out_tab, comb_h, nidx_h, scp_h,
               comb_v, nidx_v, iv_v, ov_v, na_v, nb_v, scp_v,
               gsem_io, gsem_ov, gsem_na, gsem_nb, ssem):
    cid = lax.axis_index("c")
    sid = lax.axis_index("s")
    wid = sid * _NC + cid

    pltpu.sync_copy(comb_h.at[pl.ds(wid * (_NBLK * 2 * _RB), _NBLK * 2 * _RB)],
                    comb_v)
    pltpu.sync_copy(nidx_h.at[pl.ds(wid * (_RPT * _S), _RPT * _S)], nidx_v)

    def issue_gathers(g, par):
        pltpu.async_copy(in_tab.at[comb_v.at[pl.ds(g * 2 * _RB, _RB)]],
                         iv_v.at[par], gsem_io)
        pltpu.async_copy(out_tab.at[comb_v.at[pl.ds(g * 2 * _RB + _RB, _RB)]],
                         ov_v.at[par], gsem_ov)
        pltpu.async_copy(out_tab.at[nidx_v.at[pl.ds(g * _RB * _S, 128)]],
                         na_v.at[par], gsem_na)
        pltpu.async_copy(out_tab.at[nidx_v.at[pl.ds(g * _RB * _S + 128, 128)]],
                         nb_v.at[par], gsem_nb)

    issue_gathers(0, 0)

    def block_body(g, carry):
        p = lax.rem(g, 2)
        q = 1 - p
        pltpu.make_async_copy(in_tab.at[pl.ds(0, _RB)], iv_v.at[p],
                              gsem_io).wait()
        pltpu.make_async_copy(out_tab.at[pl.ds(0, _RB)], ov_v.at[p],
                              gsem_ov).wait()
        pltpu.make_async_copy(out_tab.at[pl.ds(0, 128)], na_v.at[p],
                              gsem_na).wait()
        pltpu.make_async_copy(out_tab.at[pl.ds(0, 128)], nb_v.at[p],
                              gsem_nb).wait()

        @pl.when(g + 1 < _NBLK)
        def _prefetch():
            issue_gathers(g + 1, q)

        @pl.when(g >= 1)
        def _drain():
            pltpu.make_async_copy(scp_v.at[0], scp_h.at[pl.ds(0, _BLK_W)],
                                  ssem).wait()

        def unpack(vi):
            hi = lax.bitcast_convert_type(vi & jnp.int32(-65536), jnp.float32)
            lo = lax.bitcast_convert_type(vi << 16, jnp.float32)
            return lo, hi

        _C = _D // 32
        for r in range(_RB):
            ich = []
            for k in range(_C):
                lo, hi = unpack(iv_v[p, r, pl.ds(k * _L, _L)])
                ich += [lo, hi]
            nbuf = na_v if r < 8 else nb_v
            for s in range(_S):
                j = (r % 8) * _S + s
                acc = None
                for k in range(_C):
                    nlo, nhi = unpack(nbuf[p, j, pl.ds(k * _L, _L)])
                    t = nlo * ich[2 * k] + nhi * ich[2 * k + 1]
                    acc = t if acc is None else acc + t
                scp_v[p, pl.ds((r * _S + s) * _L, _L)] = -acc
            pacc = None
            for k in range(_C):
                olo, ohi = unpack(ov_v[p, r, pl.ds(k * _L, _L)])
                t = olo * ich[2 * k] + ohi * ich[2 * k + 1]
                pacc = t if pacc is None else pacc + t
            scp_v[p, pl.ds(_RB * _S * _L + r * _L, _L)] = pacc

        gb = wid * _NBLK + g
        pltpu.async_copy(scp_v.at[p], scp_h.at[pl.ds(gb * _BLK_W, _BLK_W)],
                         ssem)
        return carry

    lax.fori_loop(0, _NBLK, block_body, 0)
    pltpu.make_async_copy(scp_v.at[1], scp_h.at[pl.ds(0, _BLK_W)], ssem).wait()


def _tc_loss(scp_ref, mask_ref, g_ref, out_ref):
    t = pl.program_id(0)

    def logsig(x):
        return jnp.minimum(x, 0.0) - jnp.log1p(jnp.exp(-jnp.abs(x)))

    y = jax.lax.dot(scp_ref[...], g_ref[...])
    contrib = jnp.sum(logsig(y) * mask_ref[...])

    @pl.when(t == 0)
    def _init():
        out_ref[...] = jnp.zeros((1, 1), jnp.float32)

    out_ref[...] = out_ref[...] + jnp.full((1, 1), contrib, jnp.float32)

    @pl.when(t == _TC_STEPS - 1)
    def _fin():
        out_ref[...] = out_ref[...] * (-1.0 / _B)


def kernel(input_labes, out_labels, num_sampled, in_embed, out_embed):
    inp_idx = jnp.tile(input_labes, _W).astype(jnp.int32)
    out_idx = out_labels.reshape(-1).astype(jnp.int32)
    comb = jnp.concatenate([inp_idx.reshape(_NBLK_G, _RB),
                            out_idx.reshape(_NBLK_G, _RB)],
                           axis=1).reshape(-1)
    noise_idx = jax.random.randint(jax.random.key(42), (_N, _S),
                                   0, _NUM_CLASSES - 1).astype(jnp.int32)
    noise_flat = noise_idx.reshape(-1)

    mesh = plsc.VectorSubcoreMesh(core_axis_name="c", subcore_axis_name="s")
    sc = functools.partial(
        pl.kernel, mesh=mesh,
        compiler_params=pltpu.CompilerParams(use_tc_tiling_on_sc=False),
        out_type=[jax.ShapeDtypeStruct((_NBLK_G * _BLK_W,), jnp.float32)],
        scratch_types=[
            pltpu.VMEM((_NBLK * 2 * _RB,), jnp.int32),
            pltpu.VMEM((_RPT * _S,), jnp.int32),
            pltpu.VMEM((2, _RB, _D // 2), jnp.int32),
            pltpu.VMEM((2, _RB, _D // 2), jnp.int32),
            pltpu.VMEM((2, 128, _D // 2), jnp.int32),
            pltpu.VMEM((2, 128, _D // 2), jnp.int32),
            pltpu.VMEM((2, _BLK_W), jnp.float32),
            pltpu.SemaphoreType.DMA,
            pltpu.SemaphoreType.DMA,
            pltpu.SemaphoreType.DMA,
            pltpu.SemaphoreType.DMA,
            pltpu.SemaphoreType.DMA,
        ],
    )(_sc_scores)
    def pack_table(t):
        tb = t.astype(jnp.bfloat16).reshape(_NUM_CLASSES, _D // 2, 2)
        return jax.lax.bitcast_convert_type(tb, jnp.int32)

    (scp,) = sc(pack_table(in_embed), pack_table(out_embed), comb, noise_flat)
    scp2 = scp.reshape(_NBLK_G * _BLK_W // 128, 128)

    gi = jnp.arange(128) // _L
    g = (gi[:, None] == gi[None, :]).astype(jnp.float32)

    row = jnp.arange(34)
    col = jnp.arange(128)
    once = (col % _L == 0)[None, :]
    sid = 8 * (row[:, None] % 2) + col[None, :] // _L
    m34 = jnp.where(row[:, None] < 32, once & (sid < num_sampled), once)
    mask_full = jnp.tile(m34.astype(jnp.float32), (_TC_ROWS // 34, 1))

    loss = pl.pallas_call(
        _tc_loss,
        grid=(_TC_STEPS,),
        in_specs=[
            pl.BlockSpec((_TC_ROWS, 128), lambda t: (t, 0)),
            pl.BlockSpec((_TC_ROWS, 128), lambda t: (0, 0)),
            pl.BlockSpec((128, 128), lambda t: (0, 0)),
        ],
        out_specs=pl.BlockSpec((1, 1), lambda t: (0, 0)),
        out_shape=jax.ShapeDtypeStruct((1, 1), jnp.float32),
    )(scp2, mask_full, g)
    return loss[0, 0]


# speedup vs baseline: 3.6203x; 3.6203x over previous
import functools

import numpy as np

import jax
import jax.numpy as jnp
from jax import lax
from jax.experimental import pallas as pl
from jax.experimental.pallas import tpu as pltpu
from jax.experimental.pallas import tpu_sc as plsc

_NUM_CLASSES = 100000
_D = 128
_B = 1024
_W = 20
_S = 16
_N = _B * _W
_NC = 2
_NSC = 16
_NW = _NC * _NSC
_RPT = _N // _NW
_L = 16

_CPT = _NUM_CLASSES // _NW
_NCK = 25
_CKC = _CPT // _NCK
_NCELL = _NW * _NCK


def _build_schedule():
    def _draw():
        return np.asarray(
            jax.random.randint(jax.random.key(42), (_N, _S), 0,
                               _NUM_CLASSES - 1, dtype=jnp.int32))

    with jax.ensure_compile_time_eval():
        try:
            with jax.default_device(jax.local_devices(backend="cpu")[0]):
                noise = _draw()
        except Exception:
            noise = _draw()
    dcls = noise.reshape(-1)
    dr = (np.arange(_N, dtype=np.int64).repeat(_S) % _B).astype(np.int32)
    dsmp = np.tile(np.arange(_S, dtype=np.int32), _N)
    order = np.argsort(dcls, kind="stable")
    c_s, r_s, s_s = dcls[order], dr[order], dsmp[order]
    cell = (c_s // _CPT) * _NCK + (c_s % _CPT) // _CKC
    j_s = (c_s % _CPT) % _CKC
    cnt = np.bincount(cell, minlength=_NCELL)
    m16 = int(((cnt.max() + 15) // 16) * 16)
    packed = np.zeros((_NCELL, m16), np.int32)
    s_pad = np.full((_NCELL, m16), _S, np.int32)
    off = np.concatenate([[0], np.cumsum(cnt)])
    for cid in range(_NCELL):
        seg = slice(off[cid], off[cid + 1])
        n = cnt[cid]
        packed[cid, :n] = (j_s[seg] << 10) | r_s[seg]
        s_pad[cid, :n] = s_s[seg]
    return m16, packed.reshape(-1), s_pad.reshape(-1)


_SCHED = None


def _get_schedule():
    global _SCHED, _M16, _PACKED_NP, _SPAD_NP, _NGRP, _DW, _DROWS
    if _SCHED is None:
        _M16, _PACKED_NP, _SPAD_NP = _build_schedule()
        _NGRP = _M16 // 16
        _DW = _M16 * _L
        _DROWS = _NCELL * _DW // 128
        _SCHED = True
    return _SCHED


_PROWS = _N * _L // 128

_TCD_STEPS = 25
_TCP_STEPS = 4


def _sc_scores(in_tab, out_tab, il_h, oidx_h, pk_h, scpd_h, scpp_h,
               il_v, oidx_v, gbuf, u_v, pk_v, ck_v, scpd_v, scpp_v, sem):
    cid = lax.axis_index("c")
    sid = lax.axis_index("s")
    w = sid * _NC + cid

    pltpu.sync_copy(il_h, il_v)
    pltpu.sync_copy(oidx_h.at[pl.ds(w * _RPT, _RPT)], oidx_v)

    mhi = jnp.int32(-65536)

    def unpack(vi):
        lo = lax.bitcast_convert_type(vi << 16, jnp.float32)
        hi = lax.bitcast_convert_type(vi & mhi, jnp.float32)
        return lo, hi

    def u_chunk(cc, carry):
        pltpu.async_copy(in_tab.at[il_v.at[pl.ds(cc * 128, 128)]], gbuf,
                         sem).wait()

        def u_row(rr, c2):
            uch = [gbuf[rr, pl.ds(k * _L, _L)] for k in range(8)]
            uoff = (cc * 128 + rr) * 64
            for k in range(4):
                lo = lax.shift_right_logical(
                    lax.bitcast_convert_type(uch[k], jnp.int32)
                    + jnp.int32(0x8000), 16)
                hi = (lax.bitcast_convert_type(uch[k + 4], jnp.int32)
                      + jnp.int32(0x8000)) & mhi
                u_v[pl.ds(uoff + k * _L, _L)] = lo | hi
            return c2

        lax.fori_loop(0, 128, u_row, 0, unroll=4)
        return carry

    lax.fori_loop(0, _B // 128, u_chunk, 0)

    def pos_blk(bb, carry):
        pltpu.async_copy(out_tab.at[oidx_v.at[pl.ds(bb * 128, 128)]], gbuf,
                         sem).wait()

        def pos_row(rr, c2):
            r = (w * _RPT + bb * 128 + rr) & (_B - 1)
            uoff = r * 64
            och = [gbuf[rr, pl.ds(k * _L, _L)] for k in range(8)]
            acc = None
            for k in range(4):
                ulo, uhi = unpack(u_v[pl.ds(uoff + k * _L, _L)])
                t = och[k] * ulo + och[k + 4] * uhi
                acc = t if acc is None else acc + t
            scpp_v[pl.ds(rr * _L, _L)] = acc
            return c2

        lax.fori_loop(0, 128, pos_row, 0, unroll=4)
        pltpu.sync_copy(scpp_v,
                        scpp_h.at[pl.ds((w * _RPT + bb * 128) * _L, 128 * _L)])
        return carry

    lax.fori_loop(0, _RPT // 128, pos_blk, 0)

    def nz_chunk(t, carry):
        cellid = w * _NCK + t
        pltpu.sync_copy(pk_h.at[pl.ds(cellid * _M16, _M16)], pk_v)
        pltpu.sync_copy(out_tab.at[pl.ds(w * _CPT + t * _CKC, _CKC)], ck_v)

        def grp(g, c2):
            gv = pk_v[pl.ds(g * 16, 16)]
            for u in range(16):
                wd = gv[u]
                j = lax.shift_right_logical(wd, 10)
                r = wd & (_B - 1)
                uoff = r * 64
                acc = None
                for k in range(4):
                    ulo, uhi = unpack(u_v[pl.ds(uoff + k * _L, _L)])
                    nlo = ck_v[j, pl.ds(k * _L, _L)]
                    nhi = ck_v[j, pl.ds(64 + k * _L, _L)]
                    tt = nlo * ulo + nhi * uhi
                    acc = tt if acc is None else acc + tt
                scpd_v[pl.ds((g * 16 + u) * _L, _L)] = -acc
            return c2

        lax.fori_loop(0, _NGRP, grp, 0)
        pltpu.sync_copy(scpd_v, scpd_h.at[pl.ds(cellid * _DW, _DW)])
        return carry

    lax.fori_loop(0, _NCK, nz_chunk, 0)


def _tc_loss(steps):
    def body(scp_ref, mask_ref, g_ref, out_ref):
        t = pl.program_id(0)

        def logsig(x):
            return jnp.minimum(x, 0.0) - jnp.log1p(jnp.exp(-jnp.abs(x)))

        y = jax.lax.dot(scp_ref[...], g_ref[...])
        contrib = jnp.sum(logsig(y) * mask_ref[...].astype(jnp.float32))

        @pl.when(t == 0)
        def _init():
            out_ref[...] = jnp.zeros((1, 1), jnp.float32)

        out_ref[...] = out_ref[...] + jnp.full((1, 1), contrib, jnp.float32)

    return body


def kernel(input_labes, out_labels, num_sampled, in_embed, out_embed):
    _get_schedule()
    il32 = input_labes.astype(jnp.int32)
    out_idx = out_labels.reshape(-1).astype(jnp.int32)
    pk = jnp.asarray(_PACKED_NP)

    mesh = plsc.VectorSubcoreMesh(core_axis_name="c", subcore_axis_name="s")
    sc = functools.partial(
        pl.kernel, mesh=mesh,
        compiler_params=pltpu.CompilerParams(use_tc_tiling_on_sc=False),
        out_type=[jax.ShapeDtypeStruct((_NCELL * _DW,), jnp.float32),
                  jax.ShapeDtypeStruct((_N * _L,), jnp.float32)],
        scratch_types=[
            pltpu.VMEM((_B,), jnp.int32),
            pltpu.VMEM((_RPT,), jnp.int32),
            pltpu.VMEM((128, _D), jnp.float32),
            pltpu.VMEM((_B * 64,), jnp.int32),
            pltpu.VMEM((_M16,), jnp.int32),
            pltpu.VMEM((_CKC, _D), jnp.float32),
            pltpu.VMEM((_DW,), jnp.float32),
            pltpu.VMEM((128 * _L,), jnp.float32),
            pltpu.SemaphoreType.DMA,
        ],
    )(_sc_scores)
    scpd, scpp = sc(in_embed, out_embed, il32, out_idx, pk)

    scpd2 = scpd.reshape(_DROWS, 128)
    scpp2 = scpp.reshape(_PROWS, 128)

    gi = jnp.arange(128) // _L
    g = (gi[:, None] == gi[None, :]).astype(jnp.float32)

    colpat = (jnp.arange(_L) == 0)
    s2 = jnp.asarray(_SPAD_NP).reshape(_DROWS, 8)
    maskd = ((s2 < num_sampled)[:, :, None] & colpat[None, None, :]) \
        .reshape(_DROWS, 128).astype(jnp.bfloat16)
    maskp = jnp.tile(colpat, 8).reshape(1, 128).astype(jnp.float32)

    sd = pl.pallas_call(
        _tc_loss(_TCD_STEPS),
        grid=(_TCD_STEPS,),
        in_specs=[
            pl.BlockSpec((_DROWS // _TCD_STEPS, 128), lambda t: (t, 0)),
            pl.BlockSpec((_DROWS // _TCD_STEPS, 128), lambda t: (t, 0)),
            pl.BlockSpec((128, 128), lambda t: (0, 0)),
        ],
        out_specs=pl.BlockSpec((1, 1), lambda t: (0, 0)),
        out_shape=jax.ShapeDtypeStruct((1, 1), jnp.float32),
    )(scpd2, maskd, g)

    sp = pl.pallas_call(
        _tc_loss(_TCP_STEPS),
        grid=(_TCP_STEPS,),
        in_specs=[
            pl.BlockSpec((_PROWS // _TCP_STEPS, 128), lambda t: (t, 0)),
            pl.BlockSpec((1, 128), lambda t: (0, 0)),
            pl.BlockSpec((128, 128), lambda t: (0, 0)),
        ],
        out_specs=pl.BlockSpec((1, 1), lambda t: (0, 0)),
        out_shape=jax.ShapeDtypeStruct((1, 1), jnp.float32),
    )(scpp2, maskp, g)

    return -(sd[0, 0] + sp[0, 0]) / _B


# speedup vs baseline: 4.1250x; 1.1394x over previous
import functools

import numpy as np

import jax
import jax.numpy as jnp
from jax import lax
from jax.experimental import pallas as pl
from jax.experimental.pallas import tpu as pltpu
from jax.experimental.pallas import tpu_sc as plsc

_NUM_CLASSES = 100000
_D = 128
_B = 1024
_W = 20
_S = 16
_N = _B * _W
_NC = 2
_NSC = 16
_NW = _NC * _NSC
_RPT = _N // _NW
_L = 16

_CPT = _NUM_CLASSES // _NW
_NCK = 25
_CKC = _CPT // _NCK
_NCELL = _NW * _NCK


def _build_schedule():
    def _draw():
        return np.asarray(
            jax.random.randint(jax.random.key(42), (_N, _S), 0,
                               _NUM_CLASSES - 1, dtype=jnp.int32))

    with jax.ensure_compile_time_eval():
        try:
            with jax.default_device(jax.local_devices(backend="cpu")[0]):
                noise = _draw()
        except Exception:
            noise = _draw()
    dcls = noise.reshape(-1)
    dr = (np.arange(_N, dtype=np.int64).repeat(_S) % _B).astype(np.int32)
    dsmp = np.tile(np.arange(_S, dtype=np.int32), _N)
    order = np.argsort(dcls, kind="stable")
    c_s, r_s, s_s = dcls[order], dr[order], dsmp[order]
    cell = (c_s // _CPT) * _NCK + (c_s % _CPT) // _CKC
    j_s = (c_s % _CPT) % _CKC
    cnt = np.bincount(cell, minlength=_NCELL)
    m16 = int(((cnt.max() + 15) // 16) * 16)
    packed = np.zeros((_NCELL, m16), np.int32)
    s_pad = np.full((_NCELL, m16), _S, np.int32)
    off = np.concatenate([[0], np.cumsum(cnt)])
    for cid in range(_NCELL):
        seg = slice(off[cid], off[cid + 1])
        n = cnt[cid]
        packed[cid, :n] = (j_s[seg] << 10) | r_s[seg]
        s_pad[cid, :n] = s_s[seg]
    return m16, packed.reshape(-1), s_pad.reshape(-1)


_SCHED = None


def _get_schedule():
    global _SCHED, _M16, _PACKED_NP, _SPAD_NP, _NGRP, _DW, _DROWS
    if _SCHED is None:
        _M16, _PACKED_NP, _SPAD_NP = _build_schedule()
        _NGRP = _M16 // 16
        _DW = _M16 * _L
        _DROWS = _NCELL * _DW // 128
        _SCHED = True
    return _SCHED


_PROWS = _N * _L // 128

_TCD_STEPS = 25
_TCP_STEPS = 4


def _sc_scores(in_tab, out_tab, il_h, oidx_h, pk_h, scpd_h, scpp_h,
               il_v, oidx_v, gbuf, u_v, pk_v, ck_v, scpd_v, scpp_v,
               sem, psem, csem, ssem):
    cid = lax.axis_index("c")
    sid = lax.axis_index("s")
    w = sid * _NC + cid

    pltpu.sync_copy(il_h, il_v)
    pltpu.sync_copy(oidx_h.at[pl.ds(w * _RPT, _RPT)], oidx_v)

    mhi = jnp.int32(-65536)

    def unpack(vi):
        lo = lax.bitcast_convert_type(vi << 16, jnp.float32)
        hi = lax.bitcast_convert_type(vi & mhi, jnp.float32)
        return lo, hi

    def u_chunk(cc, carry):
        pltpu.async_copy(in_tab.at[il_v.at[pl.ds(cc * 128, 128)]], gbuf,
                         sem).wait()

        def u_row(rr, c2):
            uch = [gbuf[rr, pl.ds(k * _L, _L)] for k in range(8)]
            uoff = (cc * 128 + rr) * 64
            for k in range(4):
                lo = lax.shift_right_logical(
                    lax.bitcast_convert_type(uch[k], jnp.int32)
                    + jnp.int32(0x8000), 16)
                hi = (lax.bitcast_convert_type(uch[k + 4], jnp.int32)
                      + jnp.int32(0x8000)) & mhi
                u_v[pl.ds(uoff + k * _L, _L)] = lo | hi
            return c2

        lax.fori_loop(0, 128, u_row, 0, unroll=4)
        return carry

    lax.fori_loop(0, _B // 128, u_chunk, 0)

    def pos_blk(bb, carry):
        pltpu.async_copy(out_tab.at[oidx_v.at[pl.ds(bb * 128, 128)]], gbuf,
                         sem).wait()

        def pos_row(rr, c2):
            r = (w * _RPT + bb * 128 + rr) & (_B - 1)
            uoff = r * 64
            och = [gbuf[rr, pl.ds(k * _L, _L)] for k in range(8)]
            acc = None
            for k in range(4):
                ulo, uhi = unpack(u_v[pl.ds(uoff + k * _L, _L)])
                t = och[k] * ulo + och[k + 4] * uhi
                acc = t if acc is None else acc + t
            scpp_v[pl.ds(rr * _L, _L)] = acc
            return c2

        lax.fori_loop(0, 128, pos_row, 0, unroll=4)
        pltpu.sync_copy(scpp_v,
                        scpp_h.at[pl.ds((w * _RPT + bb * 128) * _L, 128 * _L)])
        return carry

    lax.fori_loop(0, _RPT // 128, pos_blk, 0)

    def issue_chunk(t, par):
        cellid = w * _NCK + t
        pltpu.async_copy(pk_h.at[pl.ds(cellid * _M16, _M16)], pk_v.at[par],
                         psem)
        pltpu.async_copy(out_tab.at[pl.ds(w * _CPT + t * _CKC, _CKC)],
                         ck_v.at[par], csem)

    issue_chunk(0, 0)

    def nz_chunk(t, carry):
        p = lax.rem(t, 2)
        q = 1 - p
        pltpu.make_async_copy(pk_h.at[pl.ds(0, _M16)], pk_v.at[p],
                              psem).wait()
        pltpu.make_async_copy(out_tab.at[pl.ds(0, _CKC)], ck_v.at[p],
                              csem).wait()

        @pl.when(t + 1 < _NCK)
        def _prefetch():
            issue_chunk(t + 1, q)

        @pl.when(t >= 1)
        def _drain():
            pltpu.make_async_copy(scpd_v, scpd_h.at[pl.ds(0, _DW)],
                                  ssem).wait()

        def grp(g, c2):
            gv = pk_v[p, pl.ds(g * 16, 16)]
            for u in range(16):
                wd = gv[u]
                j = lax.shift_right_logical(wd, 10)
                r = wd & (_B - 1)
                uoff = r * 64
                acc = None
                for k in range(4):
                    ulo, uhi = unpack(u_v[pl.ds(uoff + k * _L, _L)])
                    nlo = ck_v[p, j, pl.ds(k * _L, _L)]
                    nhi = ck_v[p, j, pl.ds(64 + k * _L, _L)]
                    tt = nlo * ulo + nhi * uhi
                    acc = tt if acc is None else acc + tt
                scpd_v[pl.ds((g * 16 + u) * _L, _L)] = -acc
            return c2

        lax.fori_loop(0, _NGRP, grp, 0)
        cellid = w * _NCK + t
        pltpu.async_copy(scpd_v, scpd_h.at[pl.ds(cellid * _DW, _DW)], ssem)
        return carry

    lax.fori_loop(0, _NCK, nz_chunk, 0)
    pltpu.make_async_copy(scpd_v, scpd_h.at[pl.ds(0, _DW)], ssem).wait()


def _tc_loss(steps):
    def body(scp_ref, mask_ref, g_ref, out_ref):
        t = pl.program_id(0)

        def logsig(x):
            return jnp.minimum(x, 0.0) - jnp.log1p(jnp.exp(-jnp.abs(x)))

        y = jax.lax.dot(scp_ref[...], g_ref[...])
        contrib = jnp.sum(logsig(y) * mask_ref[...].astype(jnp.float32))

        @pl.when(t == 0)
        def _init():
            out_ref[...] = jnp.zeros((1, 1), jnp.float32)

        out_ref[...] = out_ref[...] + jnp.full((1, 1), contrib, jnp.float32)

    return body


def kernel(input_labes, out_labels, num_sampled, in_embed, out_embed):
    _get_schedule()
    il32 = input_labes.astype(jnp.int32)
    out_idx = out_labels.reshape(-1).astype(jnp.int32)
    pk = jnp.asarray(_PACKED_NP)

    mesh = plsc.VectorSubcoreMesh(core_axis_name="c", subcore_axis_name="s")
    sc = functools.partial(
        pl.kernel, mesh=mesh,
        compiler_params=pltpu.CompilerParams(use_tc_tiling_on_sc=False),
        out_type=[jax.ShapeDtypeStruct((_NCELL * _DW,), jnp.float32),
                  jax.ShapeDtypeStruct((_N * _L,), jnp.float32)],
        scratch_types=[
            pltpu.VMEM((_B,), jnp.int32),
            pltpu.VMEM((_RPT,), jnp.int32),
            pltpu.VMEM((128, _D), jnp.float32),
            pltpu.VMEM((_B * 64,), jnp.int32),
            pltpu.VMEM((2, _M16), jnp.int32),
            pltpu.VMEM((2, _CKC, _D), jnp.float32),
            pltpu.VMEM((_DW,), jnp.float32),
            pltpu.VMEM((128 * _L,), jnp.float32),
            pltpu.SemaphoreType.DMA,
            pltpu.SemaphoreType.DMA,
            pltpu.SemaphoreType.DMA,
            pltpu.SemaphoreType.DMA,
        ],
    )(_sc_scores)
    scpd, scpp = sc(in_embed, out_embed, il32, out_idx, pk)

    scpd2 = scpd.reshape(_DROWS, 128)
    scpp2 = scpp.reshape(_PROWS, 128)

    gi = jnp.arange(128) // _L
    g = (gi[:, None] == gi[None, :]).astype(jnp.float32)

    colpat = (jnp.arange(_L) == 0)
    s2 = jnp.asarray(_SPAD_NP).reshape(_DROWS, 8)
    maskd = ((s2 < num_sampled)[:, :, None] & colpat[None, None, :]) \
        .reshape(_DROWS, 128).astype(jnp.bfloat16)
    maskp = jnp.tile(colpat, 8).reshape(1, 128).astype(jnp.float32)

    sd = pl.pallas_call(
        _tc_loss(_TCD_STEPS),
        grid=(_TCD_STEPS,),
        in_specs=[
            pl.BlockSpec((_DROWS // _TCD_STEPS, 128), lambda t: (t, 0)),
            pl.BlockSpec((_DROWS // _TCD_STEPS, 128), lambda t: (t, 0)),
            pl.BlockSpec((128, 128), lambda t: (0, 0)),
        ],
        out_specs=pl.BlockSpec((1, 1), lambda t: (0, 0)),
        out_shape=jax.ShapeDtypeStruct((1, 1), jnp.float32),
    )(scpd2, maskd, g)

    sp = pl.pallas_call(
        _tc_loss(_TCP_STEPS),
        grid=(_TCP_STEPS,),
        in_specs=[
            pl.BlockSpec((_PROWS // _TCP_STEPS, 128), lambda t: (t, 0)),
            pl.BlockSpec((1, 128), lambda t: (0, 0)),
            pl.BlockSpec((128, 128), lambda t: (0, 0)),
        ],
        out_specs=pl.BlockSpec((1, 1), lambda t: (0, 0)),
        out_shape=jax.ShapeDtypeStruct((1, 1), jnp.float32),
    )(scpp2, maskp, g)

    return -(sd[0, 0] + sp[0, 0]) / _B


# speedup vs baseline: 4.5333x; 1.0990x over previous
import functools

import numpy as np

import jax
import jax.numpy as jnp
from jax import lax
from jax.experimental import pallas as pl
from jax.experimental.pallas import tpu as pltpu
from jax.experimental.pallas import tpu_sc as plsc

_NUM_CLASSES = 100000
_D = 128
_B = 1024
_W = 20
_S = 16
_N = _B * _W
_NC = 2
_NSC = 16
_NW = _NC * _NSC
_RPT = _N // _NW
_L = 16

_CPT = _NUM_CLASSES // _NW
_NCK = 25
_CKC = _CPT // _NCK
_NCELL = _NW * _NCK


def _build_schedule():
    def _draw():
        return np.asarray(
            jax.random.randint(jax.random.key(42), (_N, _S), 0,
                               _NUM_CLASSES - 1, dtype=jnp.int32))

    with jax.ensure_compile_time_eval():
        try:
            with jax.default_device(jax.local_devices(backend="cpu")[0]):
                noise = _draw()
        except Exception:
            noise = _draw()
    dcls = noise.reshape(-1)
    dr = (np.arange(_N, dtype=np.int64).repeat(_S) % _B).astype(np.int32)
    dsmp = np.tile(np.arange(_S, dtype=np.int32), _N)
    order = np.argsort(dcls, kind="stable")
    c_s, r_s, s_s = dcls[order], dr[order], dsmp[order]
    cell = (c_s // _CPT) * _NCK + (c_s % _CPT) // _CKC
    j_s = (c_s % _CPT) % _CKC
    cnt = np.bincount(cell, minlength=_NCELL)
    m16 = int(((cnt.max() + 15) // 16) * 16)
    packed = np.zeros((_NCELL, 16 + m16), np.int32)
    s_pad = np.full((_NCELL, m16), _S, np.int32)
    off = np.concatenate([[0], np.cumsum(cnt)])
    for cid in range(_NCELL):
        seg = slice(off[cid], off[cid + 1])
        n = cnt[cid]
        packed[cid, 0] = (n + 15) // 16
        packed[cid, 16:16 + n] = (j_s[seg] << 10) | r_s[seg]
        s_pad[cid, :n] = s_s[seg]
    return m16, packed.reshape(-1), s_pad.reshape(-1)


_SCHED = None


def _get_schedule():
    global _SCHED, _M16, _PACKED_NP, _SPAD_NP, _M16H, _DW, _DROWS
    if _SCHED is None:
        _M16, _PACKED_NP, _SPAD_NP = _build_schedule()
        _M16H = _M16 + 16
        _DW = _M16 * _L
        _DROWS = _NCELL * _DW // 128
        _SCHED = True
    return _SCHED


_PROWS = _N * _L // 128

_TCD_STEPS = 25
_TCP_STEPS = 4


def _sc_scores(in_tab, out_tab, il_h, oidx_h, pk_h, scpd_h, scpp_h,
               il_v, oidx_v, gbuf, u_v, pk_v, ck_v, scpd_v, scpp_v,
               sem, psem, csem, ssem):
    cid = lax.axis_index("c")
    sid = lax.axis_index("s")
    w = sid * _NC + cid

    pltpu.sync_copy(il_h, il_v)
    pltpu.sync_copy(oidx_h.at[pl.ds(w * _RPT, _RPT)], oidx_v)

    mhi = jnp.int32(-65536)

    def unpack(vi):
        lo = lax.bitcast_convert_type(vi << 16, jnp.float32)
        hi = lax.bitcast_convert_type(vi & mhi, jnp.float32)
        return lo, hi

    pltpu.async_copy(in_tab.at[il_v.at[pl.ds(0, 64)]], gbuf.at[0], sem)

    def u_chunk(cc, carry):
        p = lax.rem(cc, 2)
        q = 1 - p
        pltpu.make_async_copy(in_tab.at[pl.ds(0, 64)], gbuf.at[p],
                              sem).wait()

        @pl.when(cc + 1 < _B // 64)
        def _pf():
            pltpu.async_copy(in_tab.at[il_v.at[pl.ds((cc + 1) * 64, 64)]],
                             gbuf.at[q], sem)

        def u_row(rr, c2):
            uch = [gbuf[p, rr, pl.ds(k * _L, _L)] for k in range(8)]
            uoff = (cc * 64 + rr) * 64
            for k in range(4):
                lo = lax.shift_right_logical(
                    lax.bitcast_convert_type(uch[k], jnp.int32)
                    + jnp.int32(0x8000), 16)
                hi = (lax.bitcast_convert_type(uch[k + 4], jnp.int32)
                      + jnp.int32(0x8000)) & mhi
                u_v[pl.ds(uoff + k * _L, _L)] = lo | hi
            return c2

        lax.fori_loop(0, 64, u_row, 0, unroll=4)
        return carry

    lax.fori_loop(0, _B // 64, u_chunk, 0)

    pltpu.async_copy(out_tab.at[oidx_v.at[pl.ds(0, 64)]], gbuf.at[0], sem)

    def pos_blk(bb, carry):
        p = lax.rem(bb, 2)
        q = 1 - p
        pltpu.make_async_copy(out_tab.at[pl.ds(0, 64)], gbuf.at[p],
                              sem).wait()

        @pl.when(bb + 1 < _RPT // 64)
        def _pf():
            pltpu.async_copy(out_tab.at[oidx_v.at[pl.ds((bb + 1) * 64, 64)]],
                             gbuf.at[q], sem)

        def pos_row(rr, c2):
            r = (w * _RPT + bb * 64 + rr) & (_B - 1)
            uoff = r * 64
            och = [gbuf[p, rr, pl.ds(k * _L, _L)] for k in range(8)]
            acc = None
            for k in range(4):
                ulo, uhi = unpack(u_v[pl.ds(uoff + k * _L, _L)])
                t = och[k] * ulo + och[k + 4] * uhi
                acc = t if acc is None else acc + t
            scpp_v[pl.ds(rr * _L, _L)] = acc
            return c2

        lax.fori_loop(0, 64, pos_row, 0, unroll=4)
        pltpu.sync_copy(scpp_v,
                        scpp_h.at[pl.ds((w * _RPT + bb * 64) * _L, 64 * _L)])
        return carry

    lax.fori_loop(0, _RPT // 64, pos_blk, 0)

    def issue_chunk(t, par):
        cellid = w * _NCK + t
        pltpu.async_copy(pk_h.at[pl.ds(cellid * _M16H, _M16H)], pk_v.at[par],
                         psem)
        pltpu.async_copy(out_tab.at[pl.ds(w * _CPT + t * _CKC, _CKC)],
                         ck_v.at[par], csem)

    issue_chunk(0, 0)

    def nz_chunk(t, carry):
        p = lax.rem(t, 2)
        q = 1 - p
        pltpu.make_async_copy(pk_h.at[pl.ds(0, _M16H)], pk_v.at[p],
                              psem).wait()
        pltpu.make_async_copy(out_tab.at[pl.ds(0, _CKC)], ck_v.at[p],
                              csem).wait()

        @pl.when(t + 1 < _NCK)
        def _prefetch():
            issue_chunk(t + 1, q)

        @pl.when(t >= 1)
        def _drain():
            pltpu.make_async_copy(scpd_v, scpd_h.at[pl.ds(0, _DW)],
                                  ssem).wait()

        ng = pk_v[p, pl.ds(0, 16)][0]

        def grp(g, c2):
            gv = pk_v[p, pl.ds(16 + g * 16, 16)]
            for u in range(16):
                wd = gv[u]
                j = lax.shift_right_logical(wd, 10)
                r = wd & (_B - 1)
                uoff = r * 64
                acc = None
                for k in range(4):
                    ulo, uhi = unpack(u_v[pl.ds(uoff + k * _L, _L)])
                    nlo = ck_v[p, j, pl.ds(k * _L, _L)]
                    nhi = ck_v[p, j, pl.ds(64 + k * _L, _L)]
                    tt = nlo * ulo + nhi * uhi
                    acc = tt if acc is None else acc + tt
                scpd_v[pl.ds((g * 16 + u) * _L, _L)] = -acc
            return c2

        lax.fori_loop(0, ng, grp, 0)
        cellid = w * _NCK + t
        pltpu.async_copy(scpd_v, scpd_h.at[pl.ds(cellid * _DW, _DW)], ssem)
        return carry

    lax.fori_loop(0, _NCK, nz_chunk, 0)
    pltpu.make_async_copy(scpd_v, scpd_h.at[pl.ds(0, _DW)], ssem).wait()


def _tc_loss(steps):
    def body(scp_ref, mask_ref, g_ref, out_ref):
        t = pl.program_id(0)

        def logsig(x):
            return jnp.minimum(x, 0.0) - jnp.log1p(jnp.exp(-jnp.abs(x)))

        y = jax.lax.dot(scp_ref[...], g_ref[...])
        contrib = jnp.sum(jnp.where(mask_ref[...] > 0, logsig(y), 0.0))

        @pl.when(t == 0)
        def _init():
            out_ref[...] = jnp.zeros((1, 1), jnp.float32)

        out_ref[...] = out_ref[...] + jnp.full((1, 1), contrib, jnp.float32)

    return body


def kernel(input_labes, out_labels, num_sampled, in_embed, out_embed):
    _get_schedule()
    il32 = input_labes.astype(jnp.int32)
    out_idx = out_labels.reshape(-1).astype(jnp.int32)
    pk = jnp.asarray(_PACKED_NP)

    mesh = plsc.VectorSubcoreMesh(core_axis_name="c", subcore_axis_name="s")
    sc = functools.partial(
        pl.kernel, mesh=mesh,
        compiler_params=pltpu.CompilerParams(use_tc_tiling_on_sc=False),
        out_type=[jax.ShapeDtypeStruct((_NCELL * _DW,), jnp.float32),
                  jax.ShapeDtypeStruct((_N * _L,), jnp.float32)],
        scratch_types=[
            pltpu.VMEM((_B,), jnp.int32),
            pltpu.VMEM((_RPT,), jnp.int32),
            pltpu.VMEM((2, 64, _D), jnp.float32),
            pltpu.VMEM((_B * 64,), jnp.int32),
            pltpu.VMEM((2, _M16H), jnp.int32),
            pltpu.VMEM((2, _CKC, _D), jnp.float32),
            pltpu.VMEM((_DW,), jnp.float32),
            pltpu.VMEM((64 * _L,), jnp.float32),
            pltpu.SemaphoreType.DMA,
            pltpu.SemaphoreType.DMA,
            pltpu.SemaphoreType.DMA,
            pltpu.SemaphoreType.DMA,
        ],
    )(_sc_scores)
    scpd, scpp = sc(in_embed, out_embed, il32, out_idx, pk)

    scpd2 = scpd.reshape(_DROWS, 128)
    scpp2 = scpp.reshape(_PROWS, 128)

    gi = jnp.arange(128) // _L
    g = (gi[:, None] == gi[None, :]).astype(jnp.float32)

    colpat = (jnp.arange(_L) == 0)
    s2 = jnp.asarray(_SPAD_NP).reshape(_DROWS, 8)
    maskd = ((s2 < num_sampled)[:, :, None] & colpat[None, None, :]) \
        .reshape(_DROWS, 128).astype(jnp.bfloat16)
    maskp = jnp.tile(colpat, 8).reshape(1, 128).astype(jnp.float32)

    sd = pl.pallas_call(
        _tc_loss(_TCD_STEPS),
        grid=(_TCD_STEPS,),
        in_specs=[
            pl.BlockSpec((_DROWS // _TCD_STEPS, 128), lambda t: (t, 0)),
            pl.BlockSpec((_DROWS // _TCD_STEPS, 128), lambda t: (t, 0)),
            pl.BlockSpec((128, 128), lambda t: (0, 0)),
        ],
        out_specs=pl.BlockSpec((1, 1), lambda t: (0, 0)),
        out_shape=jax.ShapeDtypeStruct((1, 1), jnp.float32),
    )(scpd2, maskd, g)

    sp = pl.pallas_call(
        _tc_loss(_TCP_STEPS),
        grid=(_TCP_STEPS,),
        in_specs=[
            pl.BlockSpec((_PROWS // _TCP_STEPS, 128), lambda t: (t, 0)),
            pl.BlockSpec((1, 128), lambda t: (0, 0)),
            pl.BlockSpec((128, 128), lambda t: (0, 0)),
        ],
        out_specs=pl.BlockSpec((1, 1), lambda t: (0, 0)),
        out_shape=jax.ShapeDtypeStruct((1, 1), jnp.float32),
    )(scpp2, maskp, g)

    return -(sd[0, 0] + sp[0, 0]) / _B


# speedup vs baseline: 4.6496x; 1.0257x over previous
import functools

import numpy as np

import jax
import jax.numpy as jnp
from jax import lax
from jax.experimental import pallas as pl
from jax.experimental.pallas import tpu as pltpu
from jax.experimental.pallas import tpu_sc as plsc

_NUM_CLASSES = 100000
_D = 128
_B = 1024
_W = 20
_S = 16
_N = _B * _W
_NC = 2
_NSC = 16
_NW = _NC * _NSC
_RPT = _N // _NW
_L = 16

_CPT = _NUM_CLASSES // _NW
_NCK = 25
_CKC = _CPT // _NCK
_NCELL = _NW * _NCK


def _build_schedule():
    def _draw():
        return np.asarray(
            jax.random.randint(jax.random.key(42), (_N, _S), 0,
                               _NUM_CLASSES - 1, dtype=jnp.int32))

    with jax.ensure_compile_time_eval():
        try:
            with jax.default_device(jax.local_devices(backend="cpu")[0]):
                noise = _draw()
        except Exception:
            noise = _draw()
    dcls = noise.reshape(-1)
    dr = (np.arange(_N, dtype=np.int64).repeat(_S) % _B).astype(np.int32)
    dsmp = np.tile(np.arange(_S, dtype=np.int32), _N)
    order = np.argsort(dcls, kind="stable")
    c_s, r_s, s_s = dcls[order], dr[order], dsmp[order]
    cell = (c_s // _CPT) * _NCK + (c_s % _CPT) // _CKC
    j_s = (c_s % _CPT) % _CKC
    cnt = np.bincount(cell, minlength=_NCELL)
    m16 = int(((cnt.max() + 15) // 16) * 16)
    packed = np.zeros((_NCELL, 16 + m16), np.int32)
    s_pad = np.full((_NCELL, m16), _S, np.int32)
    off = np.concatenate([[0], np.cumsum(cnt)])
    for cid in range(_NCELL):
        seg = slice(off[cid], off[cid + 1])
        n = cnt[cid]
        packed[cid, 0] = (n + 15) // 16
        packed[cid, 16:16 + n] = (j_s[seg] << 10) | r_s[seg]
        s_pad[cid, :n] = s_s[seg]
    return m16, packed.reshape(-1), s_pad.reshape(-1)


_SCHED = None


def _get_schedule():
    global _SCHED, _M16, _PACKED_NP, _SPAD_NP, _M16H, _DW, _DROWS
    if _SCHED is None:
        _M16, _PACKED_NP, _SPAD_NP = _build_schedule()
        _M16H = _M16 + 16
        _DW = _M16 * _L
        _DROWS = _NCELL * _DW // 128
        _SCHED = True
    return _SCHED


_PROWS = _N * _L // 128

_TCD_STEPS = 25
_TCP_STEPS = 4


def _sc_scores(in_tab, out_tab, il_h, oidx_h, pk_h, scpd_h, scpp_h,
               il_v, oidx_v, gbuf, u_v, pk_v, ck_v, scpd_v, scpp_v,
               sem, psem, csem, ssem):
    cid = lax.axis_index("c")
    sid = lax.axis_index("s")
    w = sid * _NC + cid

    pltpu.sync_copy(il_h, il_v)
    pltpu.sync_copy(oidx_h.at[pl.ds(w * _RPT, _RPT)], oidx_v)

    mhi = jnp.int32(-65536)

    def unpack(vi):
        lo = lax.bitcast_convert_type(vi << 16, jnp.float32)
        hi = lax.bitcast_convert_type(vi & mhi, jnp.float32)
        return lo, hi

    pltpu.async_copy(in_tab.at[il_v.at[pl.ds(0, 64)]], gbuf.at[0], sem)

    def u_chunk(cc, carry):
        p = lax.rem(cc, 2)
        q = 1 - p
        pltpu.make_async_copy(in_tab.at[pl.ds(0, 64)], gbuf.at[p],
                              sem).wait()

        @pl.when(cc + 1 < _B // 64)
        def _pf():
            pltpu.async_copy(in_tab.at[il_v.at[pl.ds((cc + 1) * 64, 64)]],
                             gbuf.at[q], sem)

        def u_row(rr, c2):
            uch = [gbuf[p, rr, pl.ds(k * _L, _L)] for k in range(8)]
            uoff = (cc * 64 + rr) * 64
            for k in range(4):
                lo = lax.shift_right_logical(
                    lax.bitcast_convert_type(uch[k], jnp.int32)
                    + jnp.int32(0x8000), 16)
                hi = (lax.bitcast_convert_type(uch[k + 4], jnp.int32)
                      + jnp.int32(0x8000)) & mhi
                u_v[pl.ds(uoff + k * _L, _L)] = lo | hi
            return c2

        lax.fori_loop(0, 64, u_row, 0, unroll=4)
        return carry

    lax.fori_loop(0, _B // 64, u_chunk, 0)

    pltpu.async_copy(out_tab.at[oidx_v.at[pl.ds(0, 64)]], gbuf.at[0], sem)

    def pos_blk(bb, carry):
        p = lax.rem(bb, 2)
        q = 1 - p
        pltpu.make_async_copy(out_tab.at[pl.ds(0, 64)], gbuf.at[p],
                              sem).wait()

        @pl.when(bb + 1 < _RPT // 64)
        def _pf():
            pltpu.async_copy(out_tab.at[oidx_v.at[pl.ds((bb + 1) * 64, 64)]],
                             gbuf.at[q], sem)

        def pos_row(rr, c2):
            r = (w * _RPT + bb * 64 + rr) & (_B - 1)
            uoff = r * 64
            och = [gbuf[p, rr, pl.ds(k * _L, _L)] for k in range(8)]
            acc = None
            for k in range(4):
                ulo, uhi = unpack(u_v[pl.ds(uoff + k * _L, _L)])
                t = och[k] * ulo + och[k + 4] * uhi
                acc = t if acc is None else acc + t
            scpp_v[pl.ds(rr * _L, _L)] = acc
            return c2

        lax.fori_loop(0, 64, pos_row, 0, unroll=4)
        pltpu.sync_copy(scpp_v,
                        scpp_h.at[pl.ds((w * _RPT + bb * 64) * _L, 64 * _L)])
        return carry

    lax.fori_loop(0, _RPT // 64, pos_blk, 0)

    def issue_chunk(t, par):
        cellid = w * _NCK + t
        pltpu.async_copy(pk_h.at[pl.ds(cellid * _M16H, _M16H)], pk_v.at[par],
                         psem)
        pltpu.async_copy(out_tab.at[pl.ds(w * _CPT + t * _CKC, _CKC)],
                         ck_v.at[par], csem)

    issue_chunk(0, 0)

    def nz_chunk(t, carry):
        p = lax.rem(t, 2)
        q = 1 - p
        pltpu.make_async_copy(pk_h.at[pl.ds(0, _M16H)], pk_v.at[p],
                              psem).wait()
        pltpu.make_async_copy(out_tab.at[pl.ds(0, _CKC)], ck_v.at[p],
                              csem).wait()

        @pl.when(t + 1 < _NCK)
        def _prefetch():
            issue_chunk(t + 1, q)

        @pl.when(t >= 1)
        def _drain():
            pltpu.make_async_copy(scpd_v, scpd_h.at[pl.ds(0, _DW)],
                                  ssem).wait()

        ng = pk_v[p, pl.ds(0, 16)][0]

        def grp(g, c2):
            gv = pk_v[p, pl.ds(16 + g * 16, 16)]
            for u in range(16):
                wd = gv[u]
                j = lax.shift_right_logical(wd, 10)
                r = wd & (_B - 1)
                uoff = r * 64
                acc = None
                for k in range(4):
                    ulo, uhi = unpack(u_v[pl.ds(uoff + k * _L, _L)])
                    nlo = ck_v[p, j, pl.ds(k * _L, _L)]
                    nhi = ck_v[p, j, pl.ds(64 + k * _L, _L)]
                    tt = nlo * ulo + nhi * uhi
                    acc = tt if acc is None else acc + tt
                scpd_v[pl.ds((g * 16 + u) * _L, _L)] = -acc
            return c2

        lax.fori_loop(0, ng, grp, 0)
        cellid = w * _NCK + t
        pltpu.async_copy(scpd_v, scpd_h.at[pl.ds(cellid * _DW, _DW)], ssem)
        return carry

    lax.fori_loop(0, _NCK, nz_chunk, 0)
    pltpu.make_async_copy(scpd_v, scpd_h.at[pl.ds(0, _DW)], ssem).wait()


def _tc_loss(steps):
    def body(scpd_ref, maskd_ref, scpp_ref, maskp_ref, g_ref, out_ref):
        t = pl.program_id(0)

        def logsig(x):
            return jnp.minimum(x, 0.0) - jnp.log1p(jnp.exp(-jnp.abs(x)))

        g = g_ref[...]
        yd = jax.lax.dot(scpd_ref[...], g)
        yp = jax.lax.dot(scpp_ref[...], g)
        contrib = jnp.sum(jnp.where(maskd_ref[...] > 0, logsig(yd), 0.0)) \
            + jnp.sum(jnp.where(maskp_ref[...] > 0, logsig(yp), 0.0))

        @pl.when(t == 0)
        def _init():
            out_ref[...] = jnp.zeros((1, 1), jnp.float32)

        out_ref[...] = out_ref[...] + jnp.full((1, 1), contrib, jnp.float32)

    return body


def kernel(input_labes, out_labels, num_sampled, in_embed, out_embed):
    _get_schedule()
    il32 = input_labes.astype(jnp.int32)
    out_idx = out_labels.reshape(-1).astype(jnp.int32)
    pk = jnp.asarray(_PACKED_NP)

    mesh = plsc.VectorSubcoreMesh(core_axis_name="c", subcore_axis_name="s")
    sc = functools.partial(
        pl.kernel, mesh=mesh,
        compiler_params=pltpu.CompilerParams(use_tc_tiling_on_sc=False),
        out_type=[jax.ShapeDtypeStruct((_NCELL * _DW,), jnp.float32),
                  jax.ShapeDtypeStruct((_N * _L,), jnp.float32)],
        scratch_types=[
            pltpu.VMEM((_B,), jnp.int32),
            pltpu.VMEM((_RPT,), jnp.int32),
            pltpu.VMEM((2, 64, _D), jnp.float32),
            pltpu.VMEM((_B * 64,), jnp.int32),
            pltpu.VMEM((2, _M16H), jnp.int32),
            pltpu.VMEM((2, _CKC, _D), jnp.float32),
            pltpu.VMEM((_DW,), jnp.float32),
            pltpu.VMEM((64 * _L,), jnp.float32),
            pltpu.SemaphoreType.DMA,
            pltpu.SemaphoreType.DMA,
            pltpu.SemaphoreType.DMA,
            pltpu.SemaphoreType.DMA,
        ],
    )(_sc_scores)
    scpd, scpp = sc(in_embed, out_embed, il32, out_idx, pk)

    scpd2 = scpd.reshape(_DROWS, 128)
    scpp2 = scpp.reshape(_PROWS, 128)

    gi = jnp.arange(128) // _L
    g = (gi[:, None] == gi[None, :]).astype(jnp.float32)

    colpat = (jnp.arange(_L) == 0)
    s2 = jnp.asarray(_SPAD_NP).reshape(_DROWS, 8)
    maskd = ((s2 < num_sampled)[:, :, None] & colpat[None, None, :]) \
        .reshape(_DROWS, 128).astype(jnp.bfloat16)
    maskp = jnp.tile(colpat, 8).reshape(1, 128).astype(jnp.float32)

    steps = 20
    tot = pl.pallas_call(
        _tc_loss(steps),
        grid=(steps,),
        in_specs=[
            pl.BlockSpec((_DROWS // steps, 128), lambda t: (t, 0)),
            pl.BlockSpec((_DROWS // steps, 128), lambda t: (t, 0)),
            pl.BlockSpec((_PROWS // steps, 128), lambda t: (t, 0)),
            pl.BlockSpec((1, 128), lambda t: (0, 0)),
            pl.BlockSpec((128, 128), lambda t: (0, 0)),
        ],
        out_specs=pl.BlockSpec((1, 1), lambda t: (0, 0)),
        out_shape=jax.ShapeDtypeStruct((1, 1), jnp.float32),
    )(scpd2, maskd, scpp2, maskp, g)

    return -tot[0, 0] / _B
